# baseline (speedup 1.0000x reference)
import functools
from math import sqrt

import jax
import jax.numpy as jnp
from jax import lax
from jax.experimental import pallas as pl
from jax.experimental.pallas import tpu as pltpu
from jax.experimental.pallas import tpu_sc as plsc

D_MODEL = 64
SCALE = sqrt(D_MODEL)

NC = 2
NS = 16
NW = NC * NS

IDX_GRP = 128
CHUNK = 512
NSUB = CHUNK // IDX_GRP
LANES = 16


def _make_sc_gather(n_total: int):
    per_w = n_total // NW
    n_chunks = per_w // CHUNK
    idx_rows_per_w = per_w // IDX_GRP

    mesh = plsc.VectorSubcoreMesh(
        core_axis_name="c", subcore_axis_name="s", num_cores=NC,
        num_subcores=NS)

    @functools.partial(
        pl.kernel,
        out_type=jax.ShapeDtypeStruct((n_total, D_MODEL), jnp.float32),
        mesh=mesh,
        compiler_params=pltpu.CompilerParams(use_tc_tiling_on_sc=False),
        scratch_types=[
            pltpu.VMEM((NSUB, IDX_GRP), jnp.int32),
            pltpu.VMEM((CHUNK, D_MODEL), jnp.float32),
            pltpu.SemaphoreType.DMA,
        ],
    )
    def gather_scale(idx_hbm, tbl_hbm, out_hbm, idx_v, rows_v, gsem):
        cid = lax.axis_index("c")
        sid = lax.axis_index("s")
        wid = sid * NC + cid

        @pl.loop(0, n_chunks)
        def _chunk(c):
            row0 = wid * idx_rows_per_w + c * NSUB
            pltpu.sync_copy(idx_hbm.at[pl.ds(row0, NSUB)], idx_v)
            for s in range(NSUB):
                pltpu.async_copy(
                    tbl_hbm.at[idx_v.at[s]],
                    rows_v.at[pl.ds(s * IDX_GRP, IDX_GRP)],
                    gsem,
                )
            for s in range(NSUB):
                pltpu.make_async_copy(
                    tbl_hbm.at[idx_v.at[s]],
                    rows_v.at[pl.ds(s * IDX_GRP, IDX_GRP)],
                    gsem,
                ).wait()

            @pl.loop(0, CHUNK)
            def _scale(i):
                for d in range(D_MODEL // LANES):
                    sl = pl.ds(d * LANES, LANES)
                    rows_v[i, sl] = rows_v[i, sl] * SCALE

            pltpu.sync_copy(
                rows_v, out_hbm.at[pl.ds(wid * per_w + c * CHUNK, CHUNK)])

    return gather_scale


def kernel(x, lut_weight):
    b, s = x.shape
    n_total = b * s
    assert n_total % (NW * CHUNK) == 0
    idx = x.reshape(n_total // IDX_GRP, IDX_GRP).astype(jnp.int32)
    out = _make_sc_gather(n_total)(idx, lut_weight)
    return out.reshape(b, s, D_MODEL)


# speedup vs baseline: 1.1398x; 1.1398x over previous
import functools
from math import sqrt

import jax
import jax.numpy as jnp
from jax import lax
from jax.experimental import pallas as pl
from jax.experimental.pallas import tpu as pltpu
from jax.experimental.pallas import tpu_sc as plsc

D_MODEL = 64
SCALE = sqrt(D_MODEL)

NC = 2
NS = 16
NW = NC * NS

GRP = 128
NBUF = 4
LANES = 16


def _make_sc_gather(n_total: int):
    per_w = n_total // NW
    n_grp = per_w // GRP

    mesh = plsc.VectorSubcoreMesh(
        core_axis_name="c", subcore_axis_name="s", num_cores=NC,
        num_subcores=NS)

    @functools.partial(
        pl.kernel,
        out_type=jax.ShapeDtypeStruct((n_total, D_MODEL), jnp.float32),
        mesh=mesh,
        compiler_params=pltpu.CompilerParams(use_tc_tiling_on_sc=False),
        scratch_types=[
            pltpu.VMEM((n_grp, GRP), jnp.int32),
            pltpu.VMEM((NBUF, GRP, D_MODEL), jnp.float32),
            pltpu.VMEM((NBUF, GRP, D_MODEL), jnp.float32),
        ] + [pltpu.SemaphoreType.DMA] * (2 * NBUF),
    )
    def gather_scale(idx_hbm, tbl_hbm, out_hbm, idx_v, rows_g, rows_s,
                     *sems):
        gsems = sems[:NBUF]
        osems = sems[NBUF:]
        cid = lax.axis_index("c")
        sid = lax.axis_index("s")
        wid = sid * NC + cid
        grp0 = wid * n_grp
        out0 = wid * per_w

        pltpu.sync_copy(idx_hbm.at[pl.ds(grp0, n_grp)], idx_v)

        def fire_gather(c, b):
            pltpu.async_copy(tbl_hbm.at[idx_v.at[c]], rows_g.at[b],
                             gsems[b])

        def wait_gather(c, b):
            pltpu.make_async_copy(tbl_hbm.at[idx_v.at[c]], rows_g.at[b],
                                  gsems[b]).wait()

        def fire_scatter(c, b):
            pltpu.async_copy(rows_s.at[b],
                             out_hbm.at[pl.ds(out0 + c * GRP, GRP)],
                             osems[b])

        def wait_scatter(c, b):
            pltpu.make_async_copy(rows_s.at[b],
                                  out_hbm.at[pl.ds(out0 + c * GRP, GRP)],
                                  osems[b]).wait()

        for b in range(NBUF):
            fire_gather(b, b)

        @pl.loop(0, n_grp, step=NBUF)
        def _round(g0):
            for b in range(NBUF):
                c = g0 + b
                wait_gather(c, b)

                @pl.when(c >= NBUF)
                def _():
                    wait_scatter(c - NBUF, b)

                @plsc.parallel_loop(0, GRP, unroll=4)
                def _scale(i):
                    for d in range(D_MODEL // LANES):
                        sl = pl.ds(d * LANES, LANES)
                        rows_s[b, i, sl] = rows_g[b, i, sl] * SCALE

                fire_scatter(c, b)

                @pl.when(c + NBUF < n_grp)
                def _():
                    fire_gather(c + NBUF, b)

        for b in range(NBUF):
            wait_scatter(n_grp - NBUF + b, b)

    return gather_scale


def kernel(x, lut_weight):
    b, s = x.shape
    n_total = b * s
    assert n_total % (NW * GRP * NBUF) == 0
    idx = x.reshape(n_total // GRP, GRP).astype(jnp.int32)
    out = _make_sc_gather(n_total)(idx, lut_weight)
    return out.reshape(b, s, D_MODEL)
